# baseline (speedup 1.0000x reference)
import functools

import jax
import jax.numpy as jnp
from jax import lax
from jax.experimental import pallas as pl
from jax.experimental.pallas import tpu as pltpu
from jax.experimental.pallas import tpu_sc as plsc

ND = 10000
NP = 10000
E = 320000
D = 128

NT = 16
C = 80
EPT = E // NT
NCH = EPT // C
GC = 10
NG = NCH // GC
NBUF = 4
RC = C
NRC = ND // RC
KPT = (NRC + NT - 1) // NT


def _sc_dual_segsum(h_drug, h_prot, didx, pidx):
    mesh = plsc.VectorSubcoreMesh(core_axis_name="c", subcore_axis_name="s")

    @functools.partial(
        pl.kernel,
        out_type=(
            jax.ShapeDtypeStruct((ND, D), jnp.float32),
            jax.ShapeDtypeStruct((NP, D), jnp.float32),
        ),
        mesh=mesh,
        scratch_types=[
            pltpu.VMEM_SHARED((ND, D), jnp.float32),
            pltpu.VMEM((2, GC, C), jnp.int32),
            pltpu.VMEM((2, GC, C), jnp.int32),
            pltpu.VMEM((NBUF, C, D), jnp.float32),
            pltpu.SemaphoreType.DMA,
            pltpu.SemaphoreType.DMA((NBUF,)),
            pltpu.SemaphoreType.DMA((NBUF,)),
        ],
    )
    def k(hd, hp, didx_h, pidx_h, p_out, q_out,
          acc, idx_s, idx_d, rows, sem_i, sem_g, sem_sc):
        cid = lax.axis_index("c")
        tid = lax.axis_index("s")

        def zero_row(r, carry):
            for cc in range(D // 16):
                rows[0, r, pl.ds(cc * 16, 16)] = jnp.zeros((16,), jnp.float32)
            return carry
        lax.fori_loop(0, RC, zero_row, 0)
        for kk in range(KPT):
            ch = kk * NT + tid

            @pl.when(ch < NRC)
            def _():
                pltpu.sync_copy(rows.at[0], acc.at[pl.ds(ch * RC, RC), :])
        plsc.subcore_barrier()

        def direction(src_tab, sidx_h, dstx_h, out_hbm):
            pltpu.async_copy(sidx_h.at[tid, 0], idx_s.at[0], sem_i)
            pltpu.async_copy(dstx_h.at[tid, 0], idx_d.at[0], sem_i)

            def group(g, carry):
                sg = g % 2
                pltpu.make_async_copy(sidx_h.at[tid, g], idx_s.at[sg],
                                      sem_i).wait()
                pltpu.make_async_copy(dstx_h.at[tid, g], idx_d.at[sg],
                                      sem_i).wait()

                @pl.when(g > 0)
                def _():
                    pltpu.make_async_copy(
                        rows.at[1], acc.at[idx_d.at[1 - sg, GC - 1]],
                        sem_sc.at[1]).wait()
                for j in range(3):
                    pltpu.async_copy(src_tab.at[idx_s.at[sg, j]],
                                     rows.at[j], sem_g.at[j])

                for j in range(GC):
                    b = j % NBUF
                    if j == 2:
                        @pl.when(g + 1 < NG)
                        def _():
                            pltpu.async_copy(sidx_h.at[tid, g + 1],
                                             idx_s.at[1 - sg], sem_i)
                            pltpu.async_copy(dstx_h.at[tid, g + 1],
                                             idx_d.at[1 - sg], sem_i)
                    if j >= 1:
                        pltpu.make_async_copy(
                            rows.at[(j - 1) % NBUF],
                            acc.at[idx_d.at[sg, j - 1]],
                            sem_sc.at[(j - 1) % NBUF]).wait()
                    if j <= GC - 4:
                        pltpu.async_copy(src_tab.at[idx_s.at[sg, j + 3]],
                                         rows.at[(j + 3) % NBUF],
                                         sem_g.at[(j + 3) % NBUF])
                    pltpu.make_async_copy(src_tab.at[idx_s.at[sg, j]],
                                          rows.at[b], sem_g.at[b]).wait()
                    pltpu.async_copy(rows.at[b], acc.at[idx_d.at[sg, j]],
                                     sem_sc.at[b], add=True)
                return carry
            lax.fori_loop(0, NG, group, 0)

            pltpu.make_async_copy(
                rows.at[(GC - 1) % NBUF],
                acc.at[idx_d.at[(NG - 1) % 2, GC - 1]],
                sem_sc.at[(GC - 1) % NBUF]).wait()
            plsc.subcore_barrier()

            for kk in range(KPT):
                ch = kk * NT + tid

                @pl.when(ch < NRC)
                def _():
                    b = kk % NBUF
                    pltpu.sync_copy(acc.at[pl.ds(ch * RC, RC), :], rows.at[b])
                    pltpu.sync_copy(rows.at[b], out_hbm.at[pl.ds(ch * RC, RC), :])

        @pl.when(cid == 0)
        def _():
            direction(hp, pidx_h, didx_h, p_out)

        @pl.when(cid == 1)
        def _():
            direction(hd, didx_h, pidx_h, q_out)

    return k(h_drug, h_prot, didx, pidx)


_ROW_SPEC = pl.BlockSpec((2000, D), lambda i: (i, 0))
_W_SPEC = pl.BlockSpec((D, D), lambda i: (0, 0))


def _tc_pre(hd, hp, w_d, w_p):
    def body(hd_r, hp_r, wd_r, wp_r, od, op):
        od[...] = jnp.dot(hd_r[...], wd_r[...],
                          preferred_element_type=jnp.float32)
        op[...] = jnp.dot(hp_r[...], wp_r[...],
                          preferred_element_type=jnp.float32)

    return pl.pallas_call(
        body,
        grid=(5,),
        in_specs=[_ROW_SPEC, _ROW_SPEC, _W_SPEC, _W_SPEC],
        out_specs=[_ROW_SPEC, _ROW_SPEC],
        out_shape=[jax.ShapeDtypeStruct((ND, D), jnp.float32)] * 2,
    )(hd, hp, w_d, w_p)


def _tc_post(pre_d, ad, pre_p, ap, w_ad, w_ap, rd=None, rp=None):
    with_res = rd is not None

    def body(*refs):
        if with_res:
            pd_r, ad_r, pp_r, ap_r, wad, wap, rd_r, rp_r, od, op = refs
        else:
            pd_r, ad_r, pp_r, ap_r, wad, wap, od, op = refs
        accd = pd_r[...] + jnp.dot(ad_r[...], wad[...],
                                   preferred_element_type=jnp.float32)
        if with_res:
            accd = accd + rd_r[...]
        od[...] = jnp.maximum(accd, 0.0)
        accp = pp_r[...] + jnp.dot(ap_r[...], wap[...],
                                   preferred_element_type=jnp.float32)
        if with_res:
            accp = accp + rp_r[...]
        op[...] = jnp.maximum(accp, 0.0)

    in_specs = [_ROW_SPEC] * 4 + [_W_SPEC] * 2 + ([_ROW_SPEC] * 2 if with_res else [])
    args = (pre_d, ad, pre_p, ap, w_ad, w_ap)
    if with_res:
        args = args + (rd, rp)
    return pl.pallas_call(
        body,
        grid=(5,),
        in_specs=in_specs,
        out_specs=[_ROW_SPEC, _ROW_SPEC],
        out_shape=[jax.ShapeDtypeStruct((ND, D), jnp.float32)] * 2,
    )(*args)


def kernel(h_drug, h_prot, edge_index,
           W1_dd, W1_pd, W1_pp, W1_dp,
           W2_dd, W2_pd, W2_pp, W2_dp):
    didx = edge_index[0].reshape(NT, NG, GC, C)
    pidx = edge_index[1].reshape(NT, NG, GC, C)
    p1, q1 = _sc_dual_segsum(h_drug, h_prot, didx, pidx)
    pre_d1, pre_p1 = _tc_pre(h_drug, h_prot, W1_dd, W1_pp)
    d1, t1 = _tc_post(pre_d1, p1, pre_p1, q1, W1_pd, W1_dp)
    p2, q2 = _sc_dual_segsum(d1, t1, didx, pidx)
    pre_d2, pre_p2 = _tc_pre(d1, t1, W2_dd, W2_pp)
    out_d, out_p = _tc_post(pre_d2, p2, pre_p2, q2, W2_pd, W2_dp,
                            rd=h_drug, rp=h_prot)
    return (out_d, out_p)


# speedup vs baseline: 1.0125x; 1.0125x over previous
import functools

import jax
import jax.numpy as jnp
from jax import lax
from jax.experimental import pallas as pl
from jax.experimental.pallas import tpu as pltpu
from jax.experimental.pallas import tpu_sc as plsc

ND = 10000
NP = 10000
E = 320000
D = 128

NT = 16
C = 80
EPT = E // NT
NCH = EPT // C
GC = 10
NG = NCH // GC
NBUF = 4
RC = C
NRC = ND // RC
KPT = (NRC + NT - 1) // NT


def _sc_dual_segsum(h_drug, h_prot, didx, pidx):
    mesh = plsc.VectorSubcoreMesh(core_axis_name="c", subcore_axis_name="s")

    @functools.partial(
        pl.kernel,
        out_type=(
            jax.ShapeDtypeStruct((ND, D), jnp.float32),
            jax.ShapeDtypeStruct((NP, D), jnp.float32),
        ),
        mesh=mesh,
        scratch_types=[
            pltpu.VMEM_SHARED((ND, D), jnp.float32),
            pltpu.VMEM((2, GC, C), jnp.int32),
            pltpu.VMEM((2, GC, C), jnp.int32),
            pltpu.VMEM((NBUF, C, D), jnp.float32),
            pltpu.SemaphoreType.DMA,
            pltpu.SemaphoreType.DMA((NBUF,)),
            pltpu.SemaphoreType.DMA((NBUF,)),
        ],
    )
    def k(hd, hp, didx_h, pidx_h, p_out, q_out,
          acc, idx_s, idx_d, rows, sem_i, sem_g, sem_sc):
        cid = lax.axis_index("c")
        tid = lax.axis_index("s")

        def zero_row(r, carry):
            for cc in range(D // 16):
                rows[0, r, pl.ds(cc * 16, 16)] = jnp.zeros((16,), jnp.float32)
            return carry
        lax.fori_loop(0, RC, zero_row, 0)
        for kk in range(KPT):
            ch = kk * NT + tid

            @pl.when(ch < NRC)
            def _():
                pltpu.async_copy(rows.at[0], acc.at[pl.ds(ch * RC, RC), :],
                                 sem_i)
        for kk in range(KPT):
            ch = kk * NT + tid

            @pl.when(ch < NRC)
            def _():
                pltpu.make_async_copy(
                    rows.at[0], acc.at[pl.ds(ch * RC, RC), :], sem_i).wait()
        plsc.subcore_barrier()

        def direction(src_tab, sidx_h, dstx_h, out_hbm):
            pltpu.async_copy(sidx_h.at[tid, 0], idx_s.at[0], sem_i)
            pltpu.async_copy(dstx_h.at[tid, 0], idx_d.at[0], sem_i)

            def group(g, carry):
                sg = g % 2
                pltpu.make_async_copy(sidx_h.at[tid, g], idx_s.at[sg],
                                      sem_i).wait()
                pltpu.make_async_copy(dstx_h.at[tid, g], idx_d.at[sg],
                                      sem_i).wait()

                @pl.when(g > 0)
                def _():
                    pltpu.make_async_copy(
                        rows.at[1], acc.at[idx_d.at[1 - sg, GC - 1]],
                        sem_sc.at[1]).wait()
                for j in range(3):
                    pltpu.async_copy(src_tab.at[idx_s.at[sg, j]],
                                     rows.at[j], sem_g.at[j])

                for j in range(GC):
                    b = j % NBUF
                    if j == 2:
                        @pl.when(g + 1 < NG)
                        def _():
                            pltpu.async_copy(sidx_h.at[tid, g + 1],
                                             idx_s.at[1 - sg], sem_i)
                            pltpu.async_copy(dstx_h.at[tid, g + 1],
                                             idx_d.at[1 - sg], sem_i)
                    if j >= 1:
                        pltpu.make_async_copy(
                            rows.at[(j - 1) % NBUF],
                            acc.at[idx_d.at[sg, j - 1]],
                            sem_sc.at[(j - 1) % NBUF]).wait()
                    if j <= GC - 4:
                        pltpu.async_copy(src_tab.at[idx_s.at[sg, j + 3]],
                                         rows.at[(j + 3) % NBUF],
                                         sem_g.at[(j + 3) % NBUF])
                    pltpu.make_async_copy(src_tab.at[idx_s.at[sg, j]],
                                          rows.at[b], sem_g.at[b]).wait()
                    pltpu.async_copy(rows.at[b], acc.at[idx_d.at[sg, j]],
                                     sem_sc.at[b], add=True)
                return carry
            lax.fori_loop(0, NG, group, 0)

            pltpu.make_async_copy(
                rows.at[(GC - 1) % NBUF],
                acc.at[idx_d.at[(NG - 1) % 2, GC - 1]],
                sem_sc.at[(GC - 1) % NBUF]).wait()
            plsc.subcore_barrier()

            for kk in range(KPT):
                ch = kk * NT + tid

                @pl.when(ch < NRC)
                def _(kk=kk, ch=ch):
                    b = kk % 2
                    if kk >= 2:
                        pltpu.make_async_copy(
                            rows.at[b],
                            out_hbm.at[pl.ds((ch - 2 * NT) * RC, RC), :],
                            sem_sc.at[b]).wait()
                    pltpu.sync_copy(acc.at[pl.ds(ch * RC, RC), :], rows.at[b])
                    pltpu.async_copy(rows.at[b],
                                     out_hbm.at[pl.ds(ch * RC, RC), :],
                                     sem_sc.at[b])
            for kk in range(KPT - 2, KPT):
                ch = kk * NT + tid

                @pl.when(ch < NRC)
                def _(kk=kk, ch=ch):
                    pltpu.make_async_copy(
                        rows.at[kk % 2],
                        out_hbm.at[pl.ds(ch * RC, RC), :],
                        sem_sc.at[kk % 2]).wait()

        @pl.when(cid == 0)
        def _():
            direction(hp, pidx_h, didx_h, p_out)

        @pl.when(cid == 1)
        def _():
            direction(hd, didx_h, pidx_h, q_out)

    return k(h_drug, h_prot, didx, pidx)


_ROW_SPEC = pl.BlockSpec((2000, D), lambda i: (i, 0))
_W_SPEC = pl.BlockSpec((D, D), lambda i: (0, 0))


def _tc_dual(hd, ad, hp, ap, w_hd, w_ad, w_hp, w_ap, rd=None, rp=None):
    with_res = rd is not None

    def body(*refs):
        if with_res:
            hd_r, ad_r, hp_r, ap_r, whd, wad, whp, wap, rd_r, rp_r, od, op = refs
        else:
            hd_r, ad_r, hp_r, ap_r, whd, wad, whp, wap, od, op = refs
        accd = (jnp.dot(hd_r[...], whd[...], preferred_element_type=jnp.float32)
                + jnp.dot(ad_r[...], wad[...], preferred_element_type=jnp.float32))
        if with_res:
            accd = accd + rd_r[...]
        od[...] = jnp.maximum(accd, 0.0)
        accp = (jnp.dot(hp_r[...], whp[...], preferred_element_type=jnp.float32)
                + jnp.dot(ap_r[...], wap[...], preferred_element_type=jnp.float32))
        if with_res:
            accp = accp + rp_r[...]
        op[...] = jnp.maximum(accp, 0.0)

    in_specs = [_ROW_SPEC] * 4 + [_W_SPEC] * 4 + ([_ROW_SPEC] * 2 if with_res else [])
    args = (hd, ad, hp, ap, w_hd, w_ad, w_hp, w_ap)
    if with_res:
        args = args + (rd, rp)
    return pl.pallas_call(
        body,
        grid=(5,),
        in_specs=in_specs,
        out_specs=[_ROW_SPEC, _ROW_SPEC],
        out_shape=[jax.ShapeDtypeStruct((ND, D), jnp.float32)] * 2,
    )(*args)


def kernel(h_drug, h_prot, edge_index,
           W1_dd, W1_pd, W1_pp, W1_dp,
           W2_dd, W2_pd, W2_pp, W2_dp):
    didx = edge_index[0].reshape(NT, NG, GC, C)
    pidx = edge_index[1].reshape(NT, NG, GC, C)
    p1, q1 = _sc_dual_segsum(h_drug, h_prot, didx, pidx)
    d1, t1 = _tc_dual(h_drug, p1, h_prot, q1, W1_dd, W1_pd, W1_pp, W1_dp)
    p2, q2 = _sc_dual_segsum(d1, t1, didx, pidx)
    out_d, out_p = _tc_dual(d1, p2, t1, q2, W2_dd, W2_pd, W2_pp, W2_dp,
                            rd=h_drug, rp=h_prot)
    return (out_d, out_p)


# speedup vs baseline: 1.0141x; 1.0015x over previous
import functools

import jax
import jax.numpy as jnp
from jax import lax
from jax.experimental import pallas as pl
from jax.experimental.pallas import tpu as pltpu
from jax.experimental.pallas import tpu_sc as plsc

ND = 10000
NP = 10000
E = 320000
D = 128

NT = 16
C = 80
EPT = E // NT
NCH = EPT // C
GC = 10
NG = NCH // GC
NBUF = 4
RC = C
NRC = ND // RC
KPT = (NRC + NT - 1) // NT


def _sc_dual_segsum(h_drug, h_prot, didx, pidx):
    mesh = plsc.VectorSubcoreMesh(core_axis_name="c", subcore_axis_name="s")

    @functools.partial(
        pl.kernel,
        out_type=(
            jax.ShapeDtypeStruct((ND, D), jnp.float32),
            jax.ShapeDtypeStruct((NP, D), jnp.float32),
        ),
        mesh=mesh,
        scratch_types=[
            pltpu.VMEM_SHARED((ND, D), jnp.float32),
            pltpu.VMEM((2, GC, C), jnp.int32),
            pltpu.VMEM((2, GC, C), jnp.int32),
            pltpu.VMEM((NBUF, C, D), jnp.float32),
            pltpu.SemaphoreType.DMA,
            pltpu.SemaphoreType.DMA((NBUF,)),
            pltpu.SemaphoreType.DMA((NBUF,)),
        ],
    )
    def k(hd, hp, didx_h, pidx_h, p_out, q_out,
          acc, idx_s, idx_d, rows, sem_i, sem_g, sem_sc):
        cid = lax.axis_index("c")
        tid = lax.axis_index("s")

        def zero_row(r, carry):
            for cc in range(D // 16):
                rows[0, r, pl.ds(cc * 16, 16)] = jnp.zeros((16,), jnp.float32)
            return carry
        lax.fori_loop(0, RC, zero_row, 0)
        for kk in range(KPT):
            ch = kk * NT + tid

            @pl.when(ch < NRC)
            def _():
                pltpu.async_copy(rows.at[0], acc.at[pl.ds(ch * RC, RC), :],
                                 sem_i)
        for kk in range(KPT):
            ch = kk * NT + tid

            @pl.when(ch < NRC)
            def _():
                pltpu.make_async_copy(
                    rows.at[0], acc.at[pl.ds(ch * RC, RC), :], sem_i).wait()
        plsc.subcore_barrier()

        def direction(src_tab, sidx_h, dstx_h, out_hbm):
            pltpu.async_copy(sidx_h.at[tid, 0], idx_s.at[0], sem_i)
            pltpu.async_copy(dstx_h.at[tid, 0], idx_d.at[0], sem_i)

            def group(g, carry):
                sg = g % 2
                pltpu.make_async_copy(sidx_h.at[tid, g], idx_s.at[sg],
                                      sem_i).wait()
                pltpu.make_async_copy(dstx_h.at[tid, g], idx_d.at[sg],
                                      sem_i).wait()

                @pl.when(g > 0)
                def _():
                    pltpu.make_async_copy(
                        rows.at[1], acc.at[idx_d.at[1 - sg, GC - 1]],
                        sem_sc.at[1]).wait()
                for j in range(3):
                    pltpu.async_copy(src_tab.at[idx_s.at[sg, j]],
                                     rows.at[j], sem_g.at[j])

                for j in range(GC):
                    b = j % NBUF
                    if j == 2:
                        @pl.when(g + 1 < NG)
                        def _():
                            pltpu.async_copy(sidx_h.at[tid, g + 1],
                                             idx_s.at[1 - sg], sem_i)
                            pltpu.async_copy(dstx_h.at[tid, g + 1],
                                             idx_d.at[1 - sg], sem_i)
                    if j >= 1:
                        pltpu.make_async_copy(
                            rows.at[(j - 1) % NBUF],
                            acc.at[idx_d.at[sg, j - 1]],
                            sem_sc.at[(j - 1) % NBUF]).wait()
                    if j <= GC - 4:
                        pltpu.async_copy(src_tab.at[idx_s.at[sg, j + 3]],
                                         rows.at[(j + 3) % NBUF],
                                         sem_g.at[(j + 3) % NBUF])
                    pltpu.make_async_copy(src_tab.at[idx_s.at[sg, j]],
                                          rows.at[b], sem_g.at[b]).wait()
                    pltpu.async_copy(rows.at[b], acc.at[idx_d.at[sg, j]],
                                     sem_sc.at[b], add=True)
                return carry
            lax.fori_loop(0, NG, group, 0)

            pltpu.make_async_copy(
                rows.at[(GC - 1) % NBUF],
                acc.at[idx_d.at[(NG - 1) % 2, GC - 1]],
                sem_sc.at[(GC - 1) % NBUF]).wait()
            plsc.subcore_barrier()

            for kk in range(KPT):
                ch = kk * NT + tid

                @pl.when(ch < NRC)
                def _(kk=kk, ch=ch):
                    b = kk % 2
                    if kk >= 2:
                        pltpu.make_async_copy(
                            rows.at[b],
                            out_hbm.at[pl.ds((ch - 2 * NT) * RC, RC), :],
                            sem_sc.at[b]).wait()
                    pltpu.sync_copy(acc.at[pl.ds(ch * RC, RC), :], rows.at[b])
                    pltpu.async_copy(rows.at[b],
                                     out_hbm.at[pl.ds(ch * RC, RC), :],
                                     sem_sc.at[b])
            for kk in range(KPT):
                ch = kk * NT + tid

                @pl.when(jnp.logical_and(ch < NRC, ch + 2 * NT >= NRC))
                def _(kk=kk, ch=ch):
                    pltpu.make_async_copy(
                        rows.at[kk % 2],
                        out_hbm.at[pl.ds(ch * RC, RC), :],
                        sem_sc.at[kk % 2]).wait()

        @pl.when(cid == 0)
        def _():
            direction(hp, pidx_h, didx_h, p_out)

        @pl.when(cid == 1)
        def _():
            direction(hd, didx_h, pidx_h, q_out)

    return k(h_drug, h_prot, didx, pidx)


_ROW_SPEC = pl.BlockSpec((2000, D), lambda i: (i, 0))
_W_SPEC = pl.BlockSpec((D, D), lambda i: (0, 0))


def _tc_dual(hd, ad, hp, ap, w_hd, w_ad, w_hp, w_ap, rd=None, rp=None):
    with_res = rd is not None

    def body(*refs):
        if with_res:
            hd_r, ad_r, hp_r, ap_r, whd, wad, whp, wap, rd_r, rp_r, od, op = refs
        else:
            hd_r, ad_r, hp_r, ap_r, whd, wad, whp, wap, od, op = refs
        accd = (jnp.dot(hd_r[...], whd[...], preferred_element_type=jnp.float32)
                + jnp.dot(ad_r[...], wad[...], preferred_element_type=jnp.float32))
        if with_res:
            accd = accd + rd_r[...]
        od[...] = jnp.maximum(accd, 0.0)
        accp = (jnp.dot(hp_r[...], whp[...], preferred_element_type=jnp.float32)
                + jnp.dot(ap_r[...], wap[...], preferred_element_type=jnp.float32))
        if with_res:
            accp = accp + rp_r[...]
        op[...] = jnp.maximum(accp, 0.0)

    in_specs = [_ROW_SPEC] * 4 + [_W_SPEC] * 4 + ([_ROW_SPEC] * 2 if with_res else [])
    args = (hd, ad, hp, ap, w_hd, w_ad, w_hp, w_ap)
    if with_res:
        args = args + (rd, rp)
    return pl.pallas_call(
        body,
        grid=(5,),
        in_specs=in_specs,
        out_specs=[_ROW_SPEC, _ROW_SPEC],
        out_shape=[jax.ShapeDtypeStruct((ND, D), jnp.float32)] * 2,
    )(*args)


def kernel(h_drug, h_prot, edge_index,
           W1_dd, W1_pd, W1_pp, W1_dp,
           W2_dd, W2_pd, W2_pp, W2_dp):
    didx = edge_index[0].reshape(NT, NG, GC, C)
    pidx = edge_index[1].reshape(NT, NG, GC, C)
    p1, q1 = _sc_dual_segsum(h_drug, h_prot, didx, pidx)
    d1, t1 = _tc_dual(h_drug, p1, h_prot, q1, W1_dd, W1_pd, W1_pp, W1_dp)
    p2, q2 = _sc_dual_segsum(d1, t1, didx, pidx)
    out_d, out_p = _tc_dual(d1, p2, t1, q2, W2_dd, W2_pd, W2_pp, W2_dp,
                            rd=h_drug, rp=h_prot)
    return (out_d, out_p)


# speedup vs baseline: 1.1035x; 1.0882x over previous
import functools

import jax
import jax.numpy as jnp
from jax import lax
from jax.experimental import pallas as pl
from jax.experimental.pallas import tpu as pltpu
from jax.experimental.pallas import tpu_sc as plsc

ND = 10000
NP = 10000
E = 320000
D = 128

NT = 16
C = 80
EPT = E // NT
NCH = EPT // C
GC = 25
NG = NCH // GC
NBUF = 3
RC = C
NRC = ND // RC
KPT = (NRC + NT - 1) // NT


def _sc_dual_segsum(h_drug, h_prot, didx, pidx):
    mesh = plsc.VectorSubcoreMesh(core_axis_name="c", subcore_axis_name="s")

    @functools.partial(
        pl.kernel,
        out_type=(
            jax.ShapeDtypeStruct((ND, D), jnp.float32),
            jax.ShapeDtypeStruct((NP, D), jnp.float32),
        ),
        mesh=mesh,
        scratch_types=[
            pltpu.VMEM_SHARED((ND, D), jnp.float32),
            pltpu.VMEM((2, GC, C), jnp.int32),
            pltpu.VMEM((2, GC, C), jnp.int32),
            pltpu.VMEM((NBUF, C, D), jnp.float32),
            pltpu.SemaphoreType.DMA,
            pltpu.SemaphoreType.DMA((NBUF,)),
            pltpu.SemaphoreType.DMA((NBUF,)),
        ],
    )
    def k(hd, hp, didx_h, pidx_h, p_out, q_out,
          acc, idx_s, idx_d, rows, sem_i, sem_g, sem_sc):
        cid = lax.axis_index("c")
        tid = lax.axis_index("s")

        def zero_row(r, carry):
            for cc in range(D // 16):
                rows[0, r, pl.ds(cc * 16, 16)] = jnp.zeros((16,), jnp.float32)
            return carry
        lax.fori_loop(0, RC, zero_row, 0)
        for kk in range(KPT):
            ch = kk * NT + tid

            @pl.when(ch < NRC)
            def _():
                pltpu.async_copy(rows.at[0], acc.at[pl.ds(ch * RC, RC), :],
                                 sem_i)
        for kk in range(KPT):
            ch = kk * NT + tid

            @pl.when(ch < NRC)
            def _():
                pltpu.make_async_copy(
                    rows.at[0], acc.at[pl.ds(ch * RC, RC), :], sem_i).wait()
        plsc.subcore_barrier()

        def direction(src_tab, sidx_h, dstx_h, out_hbm):
            pltpu.async_copy(sidx_h.at[tid, 0], idx_s.at[0], sem_i)
            pltpu.async_copy(dstx_h.at[tid, 0], idx_d.at[0], sem_i)

            def group(g, carry):
                sg = g % 2
                pltpu.make_async_copy(sidx_h.at[tid, g], idx_s.at[sg],
                                      sem_i).wait()
                pltpu.make_async_copy(dstx_h.at[tid, g], idx_d.at[sg],
                                      sem_i).wait()

                @pl.when(g > 0)
                def _():
                    pltpu.make_async_copy(
                        rows.at[(GC - 1) % NBUF],
                        acc.at[idx_d.at[1 - sg, GC - 1]],
                        sem_sc.at[(GC - 1) % NBUF]).wait()
                for j in range(2):
                    pltpu.async_copy(src_tab.at[idx_s.at[sg, j]],
                                     rows.at[j], sem_g.at[j])

                for j in range(GC):
                    b = j % NBUF
                    if j == 2:
                        @pl.when(g + 1 < NG)
                        def _():
                            pltpu.async_copy(sidx_h.at[tid, g + 1],
                                             idx_s.at[1 - sg], sem_i)
                            pltpu.async_copy(dstx_h.at[tid, g + 1],
                                             idx_d.at[1 - sg], sem_i)
                    if j >= 1:
                        pltpu.make_async_copy(
                            rows.at[(j - 1) % NBUF],
                            acc.at[idx_d.at[sg, j - 1]],
                            sem_sc.at[(j - 1) % NBUF]).wait()
                    if j <= GC - 3:
                        pltpu.async_copy(src_tab.at[idx_s.at[sg, j + 2]],
                                         rows.at[(j + 2) % NBUF],
                                         sem_g.at[(j + 2) % NBUF])
                    pltpu.make_async_copy(src_tab.at[idx_s.at[sg, j]],
                                          rows.at[b], sem_g.at[b]).wait()
                    pltpu.async_copy(rows.at[b], acc.at[idx_d.at[sg, j]],
                                     sem_sc.at[b], add=True)
                return carry
            lax.fori_loop(0, NG, group, 0)

            pltpu.make_async_copy(
                rows.at[(GC - 1) % NBUF],
                acc.at[idx_d.at[(NG - 1) % 2, GC - 1]],
                sem_sc.at[(GC - 1) % NBUF]).wait()
            plsc.subcore_barrier()

            for kk in range(KPT):
                ch = kk * NT + tid

                @pl.when(ch < NRC)
                def _(kk=kk, ch=ch):
                    b = kk % 2
                    if kk >= 2:
                        pltpu.make_async_copy(
                            rows.at[b],
                            out_hbm.at[pl.ds((ch - 2 * NT) * RC, RC), :],
                            sem_sc.at[b]).wait()
                    pltpu.sync_copy(acc.at[pl.ds(ch * RC, RC), :], rows.at[b])
                    pltpu.async_copy(rows.at[b],
                                     out_hbm.at[pl.ds(ch * RC, RC), :],
                                     sem_sc.at[b])
            for kk in range(KPT):
                ch = kk * NT + tid

                @pl.when(jnp.logical_and(ch < NRC, ch + 2 * NT >= NRC))
                def _(kk=kk, ch=ch):
                    pltpu.make_async_copy(
                        rows.at[kk % 2],
                        out_hbm.at[pl.ds(ch * RC, RC), :],
                        sem_sc.at[kk % 2]).wait()

        @pl.when(cid == 0)
        def _():
            direction(hp, pidx_h, didx_h, p_out)

        @pl.when(cid == 1)
        def _():
            direction(hd, didx_h, pidx_h, q_out)

    return k(h_drug, h_prot, didx, pidx)


_ROW_SPEC = pl.BlockSpec((2000, D), lambda i: (i, 0))
_W_SPEC = pl.BlockSpec((D, D), lambda i: (0, 0))


def _tc_dual(hd, ad, hp, ap, w_hd, w_ad, w_hp, w_ap, rd=None, rp=None):
    with_res = rd is not None

    def body(*refs):
        if with_res:
            hd_r, ad_r, hp_r, ap_r, whd, wad, whp, wap, rd_r, rp_r, od, op = refs
        else:
            hd_r, ad_r, hp_r, ap_r, whd, wad, whp, wap, od, op = refs
        accd = (jnp.dot(hd_r[...], whd[...], preferred_element_type=jnp.float32)
                + jnp.dot(ad_r[...], wad[...], preferred_element_type=jnp.float32))
        if with_res:
            accd = accd + rd_r[...]
        od[...] = jnp.maximum(accd, 0.0)
        accp = (jnp.dot(hp_r[...], whp[...], preferred_element_type=jnp.float32)
                + jnp.dot(ap_r[...], wap[...], preferred_element_type=jnp.float32))
        if with_res:
            accp = accp + rp_r[...]
        op[...] = jnp.maximum(accp, 0.0)

    in_specs = [_ROW_SPEC] * 4 + [_W_SPEC] * 4 + ([_ROW_SPEC] * 2 if with_res else [])
    args = (hd, ad, hp, ap, w_hd, w_ad, w_hp, w_ap)
    if with_res:
        args = args + (rd, rp)
    return pl.pallas_call(
        body,
        grid=(5,),
        in_specs=in_specs,
        out_specs=[_ROW_SPEC, _ROW_SPEC],
        out_shape=[jax.ShapeDtypeStruct((ND, D), jnp.float32)] * 2,
    )(*args)


def kernel(h_drug, h_prot, edge_index,
           W1_dd, W1_pd, W1_pp, W1_dp,
           W2_dd, W2_pd, W2_pp, W2_dp):
    didx = edge_index[0].reshape(NT, NG, GC, C)
    pidx = edge_index[1].reshape(NT, NG, GC, C)
    p1, q1 = _sc_dual_segsum(h_drug, h_prot, didx, pidx)
    d1, t1 = _tc_dual(h_drug, p1, h_prot, q1, W1_dd, W1_pd, W1_pp, W1_dp)
    p2, q2 = _sc_dual_segsum(d1, t1, didx, pidx)
    out_d, out_p = _tc_dual(d1, p2, t1, q2, W2_dd, W2_pd, W2_pp, W2_dp,
                            rd=h_drug, rp=h_prot)
    return (out_d, out_p)
